# baseline (speedup 1.0000x reference)
import jax
import jax.numpy as jnp
from jax.experimental import pallas as pl
from jax.experimental.pallas import tpu as pltpu

_N = 883
_K = 64
_TID = 10
_DIW = 2
_D = 1 + _K + _TID + _DIW
_TOD_MOD = 12 * 24


def _assemble_kernel(tod_ref, dow_ref, x_ref, sp_ref, tid_ref, diw_ref, out_ref):
    i = pl.program_id(0)
    tod = tod_ref[i]
    dow = dow_ref[i]
    tid_row = tid_ref[pl.ds(tod, 1), :]
    diw_row = diw_ref[pl.ds(dow, 1), :]
    xb = x_ref[0]
    sp = sp_ref[:, :]
    tidb = jnp.broadcast_to(tid_row, (_N, _TID))
    diwb = jnp.broadcast_to(diw_row, (_N, _DIW))
    out_ref[0] = jnp.concatenate([xb, sp, tidb, diwb], axis=-1)


def kernel(x, t_list, spatial_emb, tid_table, diw_table):
    b, t = x.shape[0], x.shape[1]
    bt = b * t
    t_flat = t_list.astype(jnp.int32).reshape(bt)
    tod = t_flat % _TOD_MOD
    dow = (t_flat // _TOD_MOD) % 7
    x_flat = x.reshape(bt, _N, 1)

    out = pl.pallas_call(
        _assemble_kernel,
        grid=(bt,),
        in_specs=[
            pl.BlockSpec(memory_space=pltpu.SMEM),
            pl.BlockSpec(memory_space=pltpu.SMEM),
            pl.BlockSpec((1, _N, 1), lambda i: (i, 0, 0)),
            pl.BlockSpec((_N, _K), lambda i: (0, 0)),
            pl.BlockSpec((_TOD_MOD, _TID), lambda i: (0, 0)),
            pl.BlockSpec((7, _DIW), lambda i: (0, 0)),
        ],
        out_specs=pl.BlockSpec((1, _N, _D), lambda i: (i, 0, 0)),
        out_shape=jax.ShapeDtypeStruct((bt, _N, _D), jnp.float32),
    )(tod, dow, x_flat, spatial_emb, tid_table, diw_table)
    return out.reshape(b, t, _N, _D)
